# Initial kernel scaffold; baseline (speedup 1.0000x reference)
import jax
import jax.numpy as jnp
from jax.experimental import pallas as pl


def kernel(boxes):
    raise NotImplementedError("write your pallas kernel here")


# speedup vs baseline: 1.9643x; 1.9643x over previous
import jax
import jax.numpy as jnp
from jax.experimental import pallas as pl

_N = 5000
_NPAD = 5120
_BI = 256
_THRESHOLD = 0.8


def _contain_kernel(rows_ref, cols_ref, out_ref):
    i = pl.program_id(0)
    x1i = rows_ref[:, 0:1]
    y1i = rows_ref[:, 1:2]
    x2i = rows_ref[:, 2:3]
    y2i = rows_ref[:, 3:4]
    x1j = cols_ref[0:1, :]
    y1j = cols_ref[1:2, :]
    x2j = cols_ref[2:3, :]
    y2j = cols_ref[3:4, :]
    aj = (x2j - x1j) * (y2j - y1j)
    m = (x1j >= x1i) & (y1j >= y1i) & (x2j <= x2i) & (y2j <= y2i)
    jj = jax.lax.broadcasted_iota(jnp.int32, (_BI, _NPAD), 1)
    ii = jax.lax.broadcasted_iota(jnp.int32, (_BI, _NPAD), 0) + i * _BI
    m = m & (jj != ii)
    s = jnp.sum(
        jnp.where(m, jnp.broadcast_to(aj, (_BI, _NPAD)), 0.0),
        axis=1,
        keepdims=True,
    )
    ai = (x2i - x1i) * (y2i - y1i)
    keep = s <= _THRESHOLD * (ai + 1e-9)
    out_ref[:, :] = rows_ref[:, :] * keep.astype(jnp.float32)


@jax.jit
def kernel(boxes):
    rows = jnp.zeros((_NPAD, 8), jnp.float32).at[:_N, :4].set(boxes)
    cols = jnp.zeros((8, _NPAD), jnp.float32).at[:4, :_N].set(boxes.T)
    out = pl.pallas_call(
        _contain_kernel,
        grid=(_NPAD // _BI,),
        in_specs=[
            pl.BlockSpec((_BI, 8), lambda i: (i, 0)),
            pl.BlockSpec((8, _NPAD), lambda i: (0, 0)),
        ],
        out_specs=pl.BlockSpec((_BI, 8), lambda i: (i, 0)),
        out_shape=jax.ShapeDtypeStruct((_NPAD, 8), jnp.float32),
    )(rows, cols)
    return out[:_N, :4]


# speedup vs baseline: 2.4227x; 1.2334x over previous
import jax
import jax.numpy as jnp
from jax.experimental import pallas as pl
from jax.experimental.pallas import tpu as pltpu

_N = 5000
_NPAD = 5120
_BI = 256
_THRESHOLD = 0.8


def _contain_kernel(rows_ref, cols_ref, out_ref):
    x1i = rows_ref[:, 0:1]
    y1i = rows_ref[:, 1:2]
    x2i = rows_ref[:, 2:3]
    y2i = rows_ref[:, 3:4]
    x1j = cols_ref[0:1, :]
    y1j = cols_ref[1:2, :]
    x2j = cols_ref[2:3, :]
    y2j = cols_ref[3:4, :]
    aj = (x2j - x1j) * (y2j - y1j)
    m = (x1j >= x1i) & (y1j >= y1i) & (x2j <= x2i) & (y2j <= y2i)
    s = jnp.sum(
        jnp.where(m, jnp.broadcast_to(aj, (_BI, _NPAD)), 0.0),
        axis=1,
        keepdims=True,
    )
    ai = (x2i - x1i) * (y2i - y1i)
    keep = (s - ai) <= _THRESHOLD * (ai + 1e-9)
    out_ref[:, :] = rows_ref[:, :] * keep.astype(jnp.float32)


@jax.jit
def kernel(boxes):
    rows = jnp.zeros((_NPAD, 8), jnp.float32).at[:_N, :4].set(boxes)
    cols = jnp.zeros((8, _NPAD), jnp.float32).at[:4, :_N].set(boxes.T)
    out = pl.pallas_call(
        _contain_kernel,
        grid=(_NPAD // _BI,),
        in_specs=[
            pl.BlockSpec((_BI, 8), lambda i: (i, 0)),
            pl.BlockSpec((8, _NPAD), lambda i: (0, 0)),
        ],
        out_specs=pl.BlockSpec((_BI, 8), lambda i: (i, 0)),
        out_shape=jax.ShapeDtypeStruct((_NPAD, 8), jnp.float32),
        compiler_params=pltpu.CompilerParams(
            dimension_semantics=("parallel",),
        ),
    )(rows, cols)
    return out[:_N, :4]


# speedup vs baseline: 2.6502x; 1.0939x over previous
import jax
import jax.numpy as jnp
from jax.experimental import pallas as pl
from jax.experimental.pallas import tpu as pltpu

_N = 5000
_NPAD = 5120
_BI = 1000
_THRESHOLD = 0.8


def _contain_kernel(rows_ref, cols_ref, out_ref):
    x1i = rows_ref[:, 0:1]
    y1i = rows_ref[:, 1:2]
    x2i = rows_ref[:, 2:3]
    y2i = rows_ref[:, 3:4]
    x1j = cols_ref[0:1, :]
    y1j = cols_ref[1:2, :]
    x2j = cols_ref[2:3, :]
    y2j = cols_ref[3:4, :]
    aj = (x2j - x1j) * (y2j - y1j)
    m = (x1j >= x1i) & (y1j >= y1i) & (x2j <= x2i) & (y2j <= y2i)
    s = jnp.sum(
        jnp.where(m, jnp.broadcast_to(aj, (_BI, _NPAD)), 0.0),
        axis=1,
        keepdims=True,
    )
    ai = (x2i - x1i) * (y2i - y1i)
    keep = (s - ai) <= _THRESHOLD * (ai + 1e-9)
    out_ref[:, :] = rows_ref[:, :] * keep.astype(jnp.float32)


@jax.jit
def kernel(boxes):
    cols = jnp.zeros((8, _NPAD), jnp.float32).at[:4, :_N].set(boxes.T)
    return pl.pallas_call(
        _contain_kernel,
        grid=(_N // _BI,),
        in_specs=[
            pl.BlockSpec((_BI, 4), lambda i: (i, 0)),
            pl.BlockSpec((8, _NPAD), lambda i: (0, 0)),
        ],
        out_specs=pl.BlockSpec((_BI, 4), lambda i: (i, 0)),
        out_shape=jax.ShapeDtypeStruct((_N, 4), jnp.float32),
        compiler_params=pltpu.CompilerParams(
            dimension_semantics=("parallel",),
        ),
    )(boxes, cols)
